# Initial kernel scaffold; baseline (speedup 1.0000x reference)
import jax
import jax.numpy as jnp
from jax.experimental import pallas as pl


def kernel(x, edge_index, e, xbatch, gamma_n, beta_n, gamma_e, beta_e, W1, as1, ad1, b1, W2, as2, ad2, b2, W3, as3, ad3, b3, Wm1, bm1, Wm2, bm2, Wm3, bm3):
    raise NotImplementedError("write your pallas kernel here")


# speedup vs baseline: 21.1493x; 21.1493x over previous
import functools

import jax
import jax.numpy as jnp
from jax import lax
from jax.experimental import pallas as pl
from jax.experimental.pallas import tpu as pltpu
from jax.experimental.pallas import tpu_sc as plsc

N = 50000
E = 800000
H = 3

NP = 50048
CH = NP // 16
E2 = E + N
B = 128
EPT = 26624
E2P = EPT * 32
NBLK = EPT // B
EPT3 = EPT * 2
NBLK3 = NBLK * 2
EPTM = 25088
EMP = EPTM * 32
NBLKM = EPTM // B

@functools.cache
def _mesh_kw():
    return dict(mesh=plsc.VectorSubcoreMesh(core_axis_name="c",
                                            subcore_axis_name="s"),
                compiler_params=pltpu.CompilerParams(needs_layout_passes=False,
                                                     use_tc_tiling_on_sc=False))
_f32 = jnp.float32
_i32 = jnp.int32


def _leaky(x, slope):
    return jnp.where(x >= 0, x, x * slope)


def _iota16():
    return lax.iota(_i32, 16)


def _c16(v, dtype=_i32):
    return jnp.full((16,), v, dtype)


def _pass_a_body(alsd, src, dst, zrow, ea_out, s_out, srcv, dstv, asrc, adst,
                 eav, acc, sem):
    c = lax.axis_index("c")
    s = lax.axis_index("s")
    wid = c * 16 + s
    pltpu.sync_copy(zrow, acc.at[pl.ds(s * CH, CH)])
    for g in range(8):
        plsc.store_scatter(eav, [_iota16() + g * 16, _c16(3)],
                           jnp.zeros((16,), _f32))
    plsc.subcore_barrier()

    base = wid * EPT

    def blk(i, carry):
        off = base + i * B
        pltpu.sync_copy(src.at[pl.ds(off, B)], srcv)
        pltpu.sync_copy(dst.at[pl.ds(off, B)], dstv)
        pltpu.async_copy(alsd.at[srcv], asrc, sem).wait()
        pltpu.async_copy(alsd.at[dstv], adst, sem).wait()
        for g in range(8):
            eid = _iota16() + g * 16
            live = (eid + off) < E2
            for h in range(H):
                a1 = plsc.load_gather(asrc, [eid, _c16(h)])
                a2 = plsc.load_gather(adst, [eid, _c16(4 + h)])
                al = _leaky(a1 + a2, 0.2)
                ea = jnp.where(live, jnp.exp(al), 0.0)
                plsc.store_scatter(eav, [eid, _c16(h)], ea)
        pltpu.sync_copy(eav, ea_out.at[pl.ds(off, B)])
        pltpu.sync_copy(eav, acc.at[dstv], add=True)
        return carry

    lax.fori_loop(0, NBLK, blk, 0)
    plsc.subcore_barrier()
    lo = s * CH
    pltpu.sync_copy(acc.at[pl.ds(lo, CH)], s_out.at[c, pl.ds(lo, CH)])


def _pass_a(alsd, src, dst, zrow4):
    f = pl.kernel(
        _pass_a_body,
        out_type=[jax.ShapeDtypeStruct((E2P, 4), _f32),
                  jax.ShapeDtypeStruct((2, NP, 4), _f32)],
        scratch_types=[pltpu.VMEM((B,), _i32), pltpu.VMEM((B,), _i32),
                       pltpu.VMEM((B, 8), _f32), pltpu.VMEM((B, 8), _f32),
                       pltpu.VMEM((B, 4), _f32),
                       pltpu.VMEM_SHARED((NP, 4), _f32),
                       pltpu.SemaphoreType.DMA],
        **_mesh_kw())
    return f(alsd, src, dst, zrow4)


def _pass_b_body(co, col_split, ea_in, s_in, htab0, htab1, src, dst, zrow,
                 acc_out, srcv, dstv, eav, srow, hrows, outr, acc, sem):
    c = lax.axis_index("c")
    s = lax.axis_index("s")
    pltpu.sync_copy(zrow, acc.at[pl.ds(s * CH, CH)])
    plsc.subcore_barrier()

    if col_split:
        base = s * EPT3
        nblk = NBLK3
    else:
        base = (c * 16 + s) * EPT
        nblk = NBLK

    def make_blk(htab):
        def blk(i, carry):
            off = base + i * B
            pltpu.sync_copy(src.at[pl.ds(off, B)], srcv)
            pltpu.sync_copy(dst.at[pl.ds(off, B)], dstv)
            pltpu.sync_copy(ea_in.at[pl.ds(off, B)], eav)
            pltpu.async_copy(s_in.at[dstv], srow, sem).wait()
            pltpu.async_copy(htab.at[srcv], hrows, sem).wait()
            for g in range(8):
                eid = _iota16() + g * 16
                coef = []
                for h in range(H):
                    eh = plsc.load_gather(eav, [eid, _c16(h)])
                    sh = plsc.load_gather(srow, [eid, _c16(h)])
                    coef.append(eh / (sh + 1e-16))
                for col in range(co):
                    v = coef[0] * plsc.load_gather(hrows, [eid, _c16(col)])
                    v += coef[1] * plsc.load_gather(hrows, [eid, _c16(co + col)])
                    v += coef[2] * plsc.load_gather(hrows, [eid, _c16(2 * co + col)])
                    plsc.store_scatter(outr, [eid, _c16(col)], v)
            pltpu.sync_copy(outr, acc.at[dstv], add=True)
            return carry
        return blk

    if col_split:
        @pl.when(c == 0)
        def _():
            lax.fori_loop(0, nblk, make_blk(htab0), 0)

        @pl.when(c == 1)
        def _():
            lax.fori_loop(0, nblk, make_blk(htab1), 0)
    else:
        lax.fori_loop(0, nblk, make_blk(htab0), 0)

    plsc.subcore_barrier()
    lo = s * CH
    pltpu.sync_copy(acc.at[pl.ds(lo, CH)], acc_out.at[c, pl.ds(lo, CH)])


def _pass_b(co, col_split, ea, s_comb, htab0, htab1, src, dst, zrow):
    body = functools.partial(_pass_b_body, co, col_split)
    f = pl.kernel(
        body,
        out_type=jax.ShapeDtypeStruct((2, NP, co), _f32),
        scratch_types=[pltpu.VMEM((B,), _i32), pltpu.VMEM((B,), _i32),
                       pltpu.VMEM((B, 4), _f32), pltpu.VMEM((B, 4), _f32),
                       pltpu.VMEM((B, H * co), _f32), pltpu.VMEM((B, co), _f32),
                       pltpu.VMEM_SHARED((NP, co), _f32),
                       pltpu.SemaphoreType.DMA],
        **_mesh_kw())
    return f(ea, s_comb, htab0, htab1, src, dst, zrow)


def _mlp_gather_body(h3f, srcm, dstm, gs_out, gd_out, idxv, rows, sem):
    wid = lax.axis_index("c") * 16 + lax.axis_index("s")
    base = wid * EPTM

    def blk(i, carry):
        off = base + i * B
        pltpu.sync_copy(srcm.at[pl.ds(off, B)], idxv)
        pltpu.async_copy(h3f.at[idxv], rows, sem).wait()
        pltpu.sync_copy(rows, gs_out.at[pl.ds(off, B)])
        pltpu.sync_copy(dstm.at[pl.ds(off, B)], idxv)
        pltpu.async_copy(h3f.at[idxv], rows, sem).wait()
        pltpu.sync_copy(rows, gd_out.at[pl.ds(off, B)])
        return carry

    lax.fori_loop(0, NBLKM, blk, 0)


def _mlp_gather(h3f, srcm, dstm):
    f = pl.kernel(
        _mlp_gather_body,
        out_type=[jax.ShapeDtypeStruct((EMP, 64), _f32),
                  jax.ShapeDtypeStruct((EMP, 64), _f32)],
        scratch_types=[pltpu.VMEM((B,), _i32), pltpu.VMEM((B, 64), _f32),
                       pltpu.SemaphoreType.DMA],
        **_mesh_kw())
    return f(h3f, srcm, dstm)


def _stats_body(x_ref, out_ref):
    @pl.when(pl.program_id(0) == 0)
    def _():
        out_ref[...] = jnp.zeros_like(out_ref)
    xv = x_ref[...]
    out_ref[0, :] += jnp.sum(xv, axis=0)
    out_ref[1, :] += jnp.sum(xv * xv, axis=0)


def _stats(x, grid):
    n, d = x.shape
    rb = n // grid
    return pl.pallas_call(
        _stats_body,
        grid=(grid,),
        in_specs=[pl.BlockSpec((rb, d), lambda i: (i, 0))],
        out_specs=pl.BlockSpec((2, d), lambda i: (0, 0)),
        out_shape=jax.ShapeDtypeStruct((2, d), _f32),
    )(x)


def _l1prep_body(x_ref, st_ref, gn_ref, bn_ref, w_ref, as_ref, h_ref, al_ref):
    st = st_ref[...]
    mu = st[0:1] * (1.0 / N)
    var = st[1:2] * (1.0 / N) - mu * mu
    xb = (x_ref[...] - mu) / jnp.sqrt(var + 1e-5) * gn_ref[...] + bn_ref[...]
    h = jnp.dot(xb, w_ref[...], preferred_element_type=_f32)
    h_ref[...] = h
    al_ref[...] = jnp.dot(h, as_ref[...], preferred_element_type=_f32)


def _l1prep(xp, xst, gn, bn, W1, AS1):
    grid = 8
    rb = NP // grid
    full = lambda *shape: pl.BlockSpec(shape, lambda i: tuple(0 for _ in shape))
    return pl.pallas_call(
        _l1prep_body,
        grid=(grid,),
        in_specs=[pl.BlockSpec((rb, 16), lambda i: (i, 0)),
                  full(2, 16), full(16), full(16),
                  full(16, 48), full(48, 8)],
        out_specs=[pl.BlockSpec((rb, 48), lambda i: (i, 0)),
                   pl.BlockSpec((rb, 8), lambda i: (i, 0))],
        out_shape=[jax.ShapeDtypeStruct((NP, 48), _f32),
                   jax.ShapeDtypeStruct((NP, 8), _f32)],
    )(xp, xst, gn, bn, W1, AS1)


def _lprep_body(acc_ref, b_ref, w_ref, as_ref, h_ref, al_ref):
    a = acc_ref[...]
    prev = (a[0] + a[1]) * (1.0 / H) + b_ref[...]
    h = jnp.dot(prev, w_ref[...], preferred_element_type=_f32)
    h_ref[...] = h
    al_ref[...] = jnp.dot(h, as_ref[...], preferred_element_type=_f32)


def _lprep(acc, b, W, AS):
    ci, hc = W.shape
    grid = 8
    rb = NP // grid
    full = lambda *shape: pl.BlockSpec(shape, lambda i: tuple(0 for _ in shape))
    return pl.pallas_call(
        _lprep_body,
        grid=(grid,),
        in_specs=[pl.BlockSpec((2, rb, ci), lambda i: (0, i, 0)),
                  full(ci), full(ci, hc), full(hc, 8)],
        out_specs=[pl.BlockSpec((rb, hc), lambda i: (i, 0)),
                   pl.BlockSpec((rb, 8), lambda i: (i, 0))],
        out_shape=[jax.ShapeDtypeStruct((NP, hc), _f32),
                   jax.ShapeDtypeStruct((NP, 8), _f32)],
    )(acc, b, W, AS)


def _l3split_body(acc_ref, b_ref, w_ref, as_ref, lo_ref, hi_ref, al_ref):
    a = acc_ref[...]
    prev = (a[0] + a[1]) * (1.0 / H) + b_ref[...]
    h = jnp.dot(prev, w_ref[...], preferred_element_type=_f32)
    al_ref[...] = jnp.dot(h, as_ref[...], preferred_element_type=_f32)
    lo_ref[...] = jnp.concatenate([h[:, 0:32], h[:, 64:96], h[:, 128:160]], 1)
    hi_ref[...] = jnp.concatenate([h[:, 32:64], h[:, 96:128], h[:, 160:192]], 1)


def _l3prep(acc, b, W, AS):
    grid = 8
    rb = NP // grid
    full = lambda *shape: pl.BlockSpec(shape, lambda i: tuple(0 for _ in shape))
    return pl.pallas_call(
        _l3split_body,
        grid=(grid,),
        in_specs=[pl.BlockSpec((2, rb, 32), lambda i: (0, i, 0)),
                  full(32), full(32, 192), full(192, 8)],
        out_specs=[pl.BlockSpec((rb, 96), lambda i: (i, 0)),
                   pl.BlockSpec((rb, 96), lambda i: (i, 0)),
                   pl.BlockSpec((rb, 8), lambda i: (i, 0))],
        out_shape=[jax.ShapeDtypeStruct((NP, 96), _f32),
                   jax.ShapeDtypeStruct((NP, 96), _f32),
                   jax.ShapeDtypeStruct((NP, 8), _f32)],
    )(acc, b, W, AS)


def _mlpprep_body(acc_ref, b_ref, out_ref):
    a = acc_ref[...]
    out_ref[...] = jnp.concatenate([a[0], a[1]], 1) * (1.0 / H) + b_ref[...]


def _mlpprep(acc3, b3):
    grid = 8
    rb = NP // grid
    full = lambda *shape: pl.BlockSpec(shape, lambda i: tuple(0 for _ in shape))
    return pl.pallas_call(
        _mlpprep_body,
        grid=(grid,),
        in_specs=[pl.BlockSpec((2, rb, 32), lambda i: (0, i, 0)), full(64)],
        out_specs=pl.BlockSpec((rb, 64), lambda i: (i, 0)),
        out_shape=jax.ShapeDtypeStruct((NP, 64), _f32),
    )(acc3, b3)


def _mlp_body(gs_ref, gd_ref, e_ref, a_ref, b_ref, c_ref, bp_ref, w2_ref,
              b2_ref, w3_ref, b3_ref, out_ref):
    z = (jnp.dot(gs_ref[...], a_ref[...], preferred_element_type=_f32)
         + jnp.dot(gd_ref[...], b_ref[...], preferred_element_type=_f32)
         + jnp.dot(e_ref[...], c_ref[...], preferred_element_type=_f32)
         + bp_ref[...])
    z = _leaky(z, 0.12)
    z = jnp.dot(z, w2_ref[...], preferred_element_type=_f32) + b2_ref[...]
    z = _leaky(z, 0.12)
    z = jnp.dot(z, w3_ref[...], preferred_element_type=_f32) + b3_ref[...]
    out_ref[...] = 1.0 / (1.0 + jnp.exp(-z))


def _mlp(gs, gd, e, A, Bm, Cp, bp, Wm2, bm2, Wm3, bm3):
    grid = 125
    rb = E // grid
    full = lambda *shape: pl.BlockSpec(shape, lambda i: tuple(0 for _ in shape))
    return pl.pallas_call(
        _mlp_body,
        grid=(grid,),
        in_specs=[pl.BlockSpec((rb, 64), lambda i: (i, 0)),
                  pl.BlockSpec((rb, 64), lambda i: (i, 0)),
                  pl.BlockSpec((rb, 10), lambda i: (i, 0)),
                  full(64, 64), full(64, 64), full(10, 64), full(64),
                  full(64, 16), full(16), full(16, 1), full(1)],
        out_specs=pl.BlockSpec((rb, 1), lambda i: (i, 0)),
        out_shape=jax.ShapeDtypeStruct((E, 1), _f32),
    )(gs, gd, e, A, Bm, Cp, bp, Wm2, bm2, Wm3, bm3)


def _make_as(a_s, a_d, co):
    AS = jnp.zeros((H * co, 8), _f32)
    for h in range(H):
        AS = AS.at[h * co:(h + 1) * co, h].set(a_s[h])
        AS = AS.at[h * co:(h + 1) * co, 4 + h].set(a_d[h])
    return AS


def kernel(x, edge_index, e, xbatch, gamma_n, beta_n, gamma_e, beta_e,
           W1, as1, ad1, b1, W2, as2, ad2, b2, W3, as3, ad3, b3,
           Wm1, bm1, Wm2, bm2, Wm3, bm3):
    src = edge_index[0]
    dst = edge_index[1]
    loop = jnp.arange(N, dtype=_i32)
    padg = jnp.full((E2P - E2,), N, _i32)
    src2 = jnp.concatenate([src, loop, padg])
    dst2 = jnp.concatenate([dst, loop, padg])
    padm = jnp.full((EMP - E,), N, _i32)
    srcm = jnp.concatenate([src, padm])
    dstm = jnp.concatenate([dst, padm])

    z4 = jnp.zeros((CH, 4), _f32)
    z16 = jnp.zeros((CH, 16), _f32)
    z32 = jnp.zeros((CH, 32), _f32)

    AS1 = _make_as(as1, ad1, 16)
    AS2 = _make_as(as2, ad2, 32)
    AS3 = _make_as(as3, ad3, 64)

    xp = jnp.concatenate([x, jnp.zeros((NP - N, 16), _f32)])
    xst = _stats(x, 10)
    est = _stats(e, 125)
    h1, alsd1 = _l1prep(xp, xst, gamma_n, beta_n, W1, AS1)
    ea1, sp1 = _pass_a(alsd1, src2, dst2, z4)
    s1 = sp1[0] + sp1[1]
    acc1 = _pass_b(16, False, ea1, s1, h1, h1, src2, dst2, z16)

    h2, alsd2 = _lprep(acc1, b1, W2, AS2)
    ea2, sp2 = _pass_a(alsd2, src2, dst2, z4)
    s2 = sp2[0] + sp2[1]
    acc2 = _pass_b(32, False, ea2, s2, h2, h2, src2, dst2, z32)

    h3lo, h3hi, alsd3 = _l3prep(acc2, b2, W3, AS3)
    ea3, sp3 = _pass_a(alsd3, src2, dst2, z4)
    s3 = sp3[0] + sp3[1]
    acc3 = _pass_b(32, True, ea3, s3, h3lo, h3hi, src2, dst2, z32)

    h3f = _mlpprep(acc3, b3)
    gs, gd = _mlp_gather(h3f, srcm, dstm)

    emu = est[0] * (1.0 / E)
    evar = est[1] * (1.0 / E) - emu * emu
    scale = gamma_e / jnp.sqrt(evar + 1e-5)
    A = Wm1[0:64]
    Bm = Wm1[64:128]
    Cp = Wm1[128:138] * scale[:, None]
    bp = bm1 + (beta_e - emu * scale) @ Wm1[128:138]
    return _mlp(gs[:E], gd[:E], e, A, Bm, Cp, bp, Wm2, bm2, Wm3, bm3)


# speedup vs baseline: 33.2551x; 1.5724x over previous
import functools

import jax
import jax.numpy as jnp
from jax import lax
from jax.experimental import pallas as pl
from jax.experimental.pallas import tpu as pltpu
from jax.experimental.pallas import tpu_sc as plsc

N = 50000
E = 800000
H = 3

NP = 50048
CH = NP // 16
E2 = E + N
B = 128
EPT = 26624
E2P = EPT * 32
NBLK = EPT // B
EPT3 = EPT * 2
NBLK3 = NBLK * 2
EPTM = 25088
EMP = EPTM * 32
NBLKM = EPTM // B
SBK = 4
SB = SBK * B

@functools.cache
def _mesh_kw():
    return dict(mesh=plsc.VectorSubcoreMesh(core_axis_name="c",
                                            subcore_axis_name="s"),
                compiler_params=pltpu.CompilerParams(needs_layout_passes=False,
                                                     use_tc_tiling_on_sc=False))
_f32 = jnp.float32
_i32 = jnp.int32


def _leaky(x, slope):
    return jnp.where(x >= 0, x, x * slope)


def _iota16():
    return lax.iota(_i32, 16)


def _c16(v, dtype=_i32):
    return jnp.full((16,), v, dtype)


def _pass_a_body(alsd, src, dst, zrow, ea_out, s_out, srcv, dstv, asrc, adst,
                 eav, acc, sem):
    c = lax.axis_index("c")
    s = lax.axis_index("s")
    wid = c * 16 + s
    pltpu.sync_copy(zrow, acc.at[pl.ds(s * CH, CH)])
    for g in range(SB // 16):
        plsc.store_scatter(eav, [_iota16() + g * 16, _c16(3)],
                           jnp.zeros((16,), _f32))
    plsc.subcore_barrier()

    base = wid * EPT
    base_r = wid * NBLK

    def blk(i, carry):
        off = base + i * SB
        row0 = base_r + i * SBK
        pltpu.sync_copy(src.at[pl.ds(row0, SBK)], srcv)
        pltpu.sync_copy(dst.at[pl.ds(row0, SBK)], dstv)
        cps = []
        for k in range(SBK):
            cps.append(pltpu.async_copy(
                alsd.at[srcv.at[k]], asrc.at[pl.ds(k * B, B)], sem))
            cps.append(pltpu.async_copy(
                alsd.at[dstv.at[k]], adst.at[pl.ds(k * B, B)], sem))
        for cp in cps:
            cp.wait()

        def grp(j, carry2):
            for gg in range(8):
                eid = _iota16() + (j * 8 + gg) * 16
                live = (eid + off) < E2
                for h in range(H):
                    a1 = plsc.load_gather(asrc, [eid, _c16(h)])
                    a2 = plsc.load_gather(adst, [eid, _c16(4 + h)])
                    al = _leaky(a1 + a2, 0.2)
                    ea = jnp.where(live, jnp.exp(al), 0.0)
                    plsc.store_scatter(eav, [eid, _c16(h)], ea)
            return carry2

        lax.fori_loop(0, SB // B, grp, 0)
        pltpu.sync_copy(eav, ea_out.at[pl.ds(off, SB)])
        for k in range(SBK):
            pltpu.sync_copy(eav.at[pl.ds(k * B, B)], acc.at[dstv.at[k]],
                            add=True)
        return carry

    lax.fori_loop(0, EPT // SB, blk, 0)
    plsc.subcore_barrier()
    lo = s * CH
    pltpu.sync_copy(acc.at[pl.ds(lo, CH)], s_out.at[c, pl.ds(lo, CH)])


def _pass_a(alsd, src, dst, zrow4):
    f = pl.kernel(
        _pass_a_body,
        out_type=[jax.ShapeDtypeStruct((E2P, 4), _f32),
                  jax.ShapeDtypeStruct((2, NP, 4), _f32)],
        scratch_types=[pltpu.VMEM((SBK, B), _i32), pltpu.VMEM((SBK, B), _i32),
                       pltpu.VMEM((SB, 8), _f32), pltpu.VMEM((SB, 8), _f32),
                       pltpu.VMEM((SB, 4), _f32),
                       pltpu.VMEM_SHARED((NP, 4), _f32),
                       pltpu.SemaphoreType.DMA],
        **_mesh_kw())
    return f(alsd, src, dst, zrow4)


def _pass_b_body(col_split, ea_in, s_in, htab0, htab1, src, dst, zrow,
                 acc_out, srcv, dstv, eav, srow, hrows, outr, acc, sem):
    c = lax.axis_index("c")
    s = lax.axis_index("s")
    pltpu.sync_copy(zrow, acc.at[pl.ds(s * CH, CH)])
    plsc.subcore_barrier()

    if col_split:
        base = s * EPT3
        base_r = s * (NBLK * 2)
        nsb = EPT3 // SB
    else:
        base = (c * 16 + s) * EPT
        base_r = (c * 16 + s) * NBLK
        nsb = EPT // SB

    def make_blk(htab):
        def blk(i, carry):
            off = base + i * SB
            row0 = base_r + i * SBK
            pltpu.sync_copy(src.at[pl.ds(row0, SBK)], srcv)
            pltpu.sync_copy(dst.at[pl.ds(row0, SBK)], dstv)
            pltpu.sync_copy(ea_in.at[pl.ds(off, SB)], eav)
            cps = []
            for k in range(SBK):
                cps.append(pltpu.async_copy(
                    s_in.at[dstv.at[k]], srow.at[pl.ds(k * B, B)], sem))
                cps.append(pltpu.async_copy(
                    htab.at[srcv.at[k]], hrows.at[pl.ds(k * B, B)], sem))
            for cp in cps:
                cp.wait()

            def grp(j, carry2):
                for gg in range(4):
                    eid = _iota16() + (j * 4 + gg) * 16
                    coef = []
                    for h in range(H):
                        eh = plsc.load_gather(eav, [eid, _c16(h)])
                        sh = plsc.load_gather(srow, [eid, _c16(h)])
                        coef.append(eh / (sh + 1e-16))
                    for col in range(16):
                        v = coef[0] * plsc.load_gather(hrows, [eid, _c16(col)])
                        v += coef[1] * plsc.load_gather(hrows,
                                                        [eid, _c16(16 + col)])
                        v += coef[2] * plsc.load_gather(hrows,
                                                        [eid, _c16(32 + col)])
                        plsc.store_scatter(outr, [eid, _c16(col)], v)
                return carry2

            lax.fori_loop(0, SB // 64, grp, 0)
            for k in range(SBK):
                pltpu.sync_copy(outr.at[pl.ds(k * B, B)], acc.at[dstv.at[k]],
                                add=True)
            return carry
        return blk

    @pl.when(c == 0)
    def _():
        lax.fori_loop(0, nsb, make_blk(htab0), 0)

    @pl.when(c == 1)
    def _():
        lax.fori_loop(0, nsb, make_blk(htab1 if col_split else htab0), 0)

    plsc.subcore_barrier()
    lo = s * CH
    pltpu.sync_copy(acc.at[pl.ds(lo, CH)], acc_out.at[c, pl.ds(lo, CH)])


def _pass_b(col_split, ea, s_comb, htab0, htab1, src, dst, zrow):
    body = functools.partial(_pass_b_body, col_split)
    f = pl.kernel(
        body,
        out_type=jax.ShapeDtypeStruct((2, NP, 16), _f32),
        scratch_types=[pltpu.VMEM((SBK, B), _i32), pltpu.VMEM((SBK, B), _i32),
                       pltpu.VMEM((SB, 4), _f32), pltpu.VMEM((SB, 4), _f32),
                       pltpu.VMEM((SB, 48), _f32), pltpu.VMEM((SB, 16), _f32),
                       pltpu.VMEM_SHARED((NP, 16), _f32),
                       pltpu.SemaphoreType.DMA],
        **_mesh_kw())
    return f(ea, s_comb, htab0, htab1, src, dst, zrow)


def _mlp_gather_body(h3f, srcm, dstm, gs_out, gd_out, idxs, idxd, rows_s,
                     rows_d, sem):
    wid = lax.axis_index("c") * 16 + lax.axis_index("s")
    base = wid * EPTM
    base_r = wid * NBLKM

    def blk(i, carry):
        off = base + i * SB
        row0 = base_r + i * SBK
        pltpu.sync_copy(srcm.at[pl.ds(row0, SBK)], idxs)
        pltpu.sync_copy(dstm.at[pl.ds(row0, SBK)], idxd)
        cps = []
        for k in range(SBK):
            cps.append(pltpu.async_copy(
                h3f.at[idxs.at[k]], rows_s.at[pl.ds(k * B, B)], sem))
            cps.append(pltpu.async_copy(
                h3f.at[idxd.at[k]], rows_d.at[pl.ds(k * B, B)], sem))
        for cp in cps:
            cp.wait()
        pltpu.sync_copy(rows_s, gs_out.at[pl.ds(off, SB)])
        pltpu.sync_copy(rows_d, gd_out.at[pl.ds(off, SB)])
        return carry

    lax.fori_loop(0, EPTM // SB, blk, 0)


def _mlp_gather(h3f, srcm, dstm):
    f = pl.kernel(
        _mlp_gather_body,
        out_type=[jax.ShapeDtypeStruct((EMP, 64), _f32),
                  jax.ShapeDtypeStruct((EMP, 64), _f32)],
        scratch_types=[pltpu.VMEM((SBK, B), _i32), pltpu.VMEM((SBK, B), _i32),
                       pltpu.VMEM((SB, 64), _f32), pltpu.VMEM((SB, 64), _f32),
                       pltpu.SemaphoreType.DMA],
        **_mesh_kw())
    return f(h3f, srcm, dstm)


def _stats_body(x_ref, out_ref):
    @pl.when(pl.program_id(0) == 0)
    def _():
        out_ref[...] = jnp.zeros_like(out_ref)
    xv = x_ref[...]
    out_ref[0, :] += jnp.sum(xv, axis=0)
    out_ref[1, :] += jnp.sum(xv * xv, axis=0)


def _stats(x, grid):
    n, d = x.shape
    rb = n // grid
    return pl.pallas_call(
        _stats_body,
        grid=(grid,),
        in_specs=[pl.BlockSpec((rb, d), lambda i: (i, 0))],
        out_specs=pl.BlockSpec((2, d), lambda i: (0, 0)),
        out_shape=jax.ShapeDtypeStruct((2, d), _f32),
    )(x)


def _l1prep_body(x_ref, st_ref, gn_ref, bn_ref, w_ref, as_ref, h_ref, al_ref):
    st = st_ref[...]
    mu = st[0:1] * (1.0 / N)
    var = st[1:2] * (1.0 / N) - mu * mu
    xb = (x_ref[...] - mu) / jnp.sqrt(var + 1e-5) * gn_ref[...] + bn_ref[...]
    h = jnp.dot(xb, w_ref[...], preferred_element_type=_f32)
    h_ref[...] = h
    al_ref[...] = jnp.dot(h, as_ref[...], preferred_element_type=_f32)


def _l1prep(xp, xst, gn, bn, W1, AS1):
    grid = 8
    rb = NP // grid
    full = lambda *shape: pl.BlockSpec(shape, lambda i: tuple(0 for _ in shape))
    return pl.pallas_call(
        _l1prep_body,
        grid=(grid,),
        in_specs=[pl.BlockSpec((rb, 16), lambda i: (i, 0)),
                  full(2, 16), full(16), full(16),
                  full(16, 48), full(48, 8)],
        out_specs=[pl.BlockSpec((rb, 48), lambda i: (i, 0)),
                   pl.BlockSpec((rb, 8), lambda i: (i, 0))],
        out_shape=[jax.ShapeDtypeStruct((NP, 48), _f32),
                   jax.ShapeDtypeStruct((NP, 8), _f32)],
    )(xp, xst, gn, bn, W1, AS1)


def _lprep_body(nsplit, combine, acc_ref, b_ref, w_ref, as_ref, *out_refs):
    a = acc_ref[...]
    if combine == "sum":
        prev = (a[0] + a[1]) * (1.0 / H) + b_ref[...]
    else:
        prev = jnp.concatenate([a[0], a[1]], 1) * (1.0 / H) + b_ref[...]
    h = jnp.dot(prev, w_ref[...], preferred_element_type=_f32)
    out_refs[-1][...] = jnp.dot(h, as_ref[...], preferred_element_type=_f32)
    for q in range(nsplit):
        out_refs[q][...] = h[:, 48 * q:48 * (q + 1)]


def _lprep(nsplit, combine, acc, b, Wp, ASp):
    ci = Wp.shape[0]
    body = functools.partial(_lprep_body, nsplit, combine)
    grid = 8
    rb = NP // grid
    full = lambda *shape: pl.BlockSpec(shape, lambda i: tuple(0 for _ in shape))
    ci_in = acc.shape[2]
    return pl.pallas_call(
        body,
        grid=(grid,),
        in_specs=[pl.BlockSpec((2, rb, ci_in), lambda i: (0, i, 0)),
                  full(ci), full(ci, 48 * nsplit), full(48 * nsplit, 8)],
        out_specs=[pl.BlockSpec((rb, 48), lambda i: (i, 0))] * nsplit
                  + [pl.BlockSpec((rb, 8), lambda i: (i, 0))],
        out_shape=[jax.ShapeDtypeStruct((NP, 48), _f32)] * nsplit
                  + [jax.ShapeDtypeStruct((NP, 8), _f32)],
    )(acc, b, Wp, ASp)


def _mlpprep_body(acca_ref, accb_ref, b_ref, out_ref):
    a = acca_ref[...]
    bb = accb_ref[...]
    out_ref[...] = (jnp.concatenate([a[0], a[1], bb[0], bb[1]], 1)
                    * (1.0 / H) + b_ref[...])


def _mlpprep(acc3a, acc3b, b3):
    grid = 8
    rb = NP // grid
    full = lambda *shape: pl.BlockSpec(shape, lambda i: tuple(0 for _ in shape))
    return pl.pallas_call(
        _mlpprep_body,
        grid=(grid,),
        in_specs=[pl.BlockSpec((2, rb, 16), lambda i: (0, i, 0)),
                  pl.BlockSpec((2, rb, 16), lambda i: (0, i, 0)), full(64)],
        out_specs=pl.BlockSpec((rb, 64), lambda i: (i, 0)),
        out_shape=jax.ShapeDtypeStruct((NP, 64), _f32),
    )(acc3a, acc3b, b3)


def _mlp_body(gs_ref, gd_ref, e_ref, a_ref, b_ref, c_ref, bp_ref, w2_ref,
              b2_ref, w3_ref, b3_ref, out_ref):
    z = (jnp.dot(gs_ref[...], a_ref[...], preferred_element_type=_f32)
         + jnp.dot(gd_ref[...], b_ref[...], preferred_element_type=_f32)
         + jnp.dot(e_ref[...], c_ref[...], preferred_element_type=_f32)
         + bp_ref[...])
    z = _leaky(z, 0.12)
    z = jnp.dot(z, w2_ref[...], preferred_element_type=_f32) + b2_ref[...]
    z = _leaky(z, 0.12)
    z = jnp.dot(z, w3_ref[...], preferred_element_type=_f32) + b3_ref[...]
    out_ref[...] = 1.0 / (1.0 + jnp.exp(-z))


def _mlp(gs, gd, e, A, Bm, Cp, bp, Wm2, bm2, Wm3, bm3):
    grid = 125
    rb = E // grid
    full = lambda *shape: pl.BlockSpec(shape, lambda i: tuple(0 for _ in shape))
    return pl.pallas_call(
        _mlp_body,
        grid=(grid,),
        in_specs=[pl.BlockSpec((rb, 64), lambda i: (i, 0)),
                  pl.BlockSpec((rb, 64), lambda i: (i, 0)),
                  pl.BlockSpec((rb, 10), lambda i: (i, 0)),
                  full(64, 64), full(64, 64), full(10, 64), full(64),
                  full(64, 16), full(16), full(16, 1), full(1)],
        out_specs=pl.BlockSpec((rb, 1), lambda i: (i, 0)),
        out_shape=jax.ShapeDtypeStruct((E, 1), _f32),
    )(gs, gd, e, A, Bm, Cp, bp, Wm2, bm2, Wm3, bm3)


def _perm(co):
    idx = [h * co + q * 16 + j
           for q in range(co // 16) for h in range(H) for j in range(16)]
    return jnp.array(idx, _i32)


def _make_as(a_s, a_d, co):
    AS = jnp.zeros((H * co, 8), _f32)
    for h in range(H):
        AS = AS.at[h * co:(h + 1) * co, h].set(a_s[h])
        AS = AS.at[h * co:(h + 1) * co, 4 + h].set(a_d[h])
    return AS


def kernel(x, edge_index, e, xbatch, gamma_n, beta_n, gamma_e, beta_e,
           W1, as1, ad1, b1, W2, as2, ad2, b2, W3, as3, ad3, b3,
           Wm1, bm1, Wm2, bm2, Wm3, bm3):
    src = edge_index[0]
    dst = edge_index[1]
    loop = jnp.arange(N, dtype=_i32)
    padg = jnp.full((E2P - E2,), N, _i32)
    src2 = jnp.concatenate([src, loop, padg]).reshape(E2P // B, B)
    dst2 = jnp.concatenate([dst, loop, padg]).reshape(E2P // B, B)
    padm = jnp.full((EMP - E,), N, _i32)
    srcm = jnp.concatenate([src, padm]).reshape(EMP // B, B)
    dstm = jnp.concatenate([dst, padm]).reshape(EMP // B, B)

    z4 = jnp.zeros((CH, 4), _f32)
    z16 = jnp.zeros((CH, 16), _f32)

    AS1 = _make_as(as1, ad1, 16)
    p2 = _perm(32)
    Wp2 = jnp.take(W2, p2, axis=1)
    ASp2 = jnp.take(_make_as(as2, ad2, 32), p2, axis=0)
    p3 = _perm(64)
    Wp3 = jnp.take(W3, p3, axis=1)
    ASp3 = jnp.take(_make_as(as3, ad3, 64), p3, axis=0)

    xp = jnp.concatenate([x, jnp.zeros((NP - N, 16), _f32)])
    xst = _stats(x, 10)
    est = _stats(e, 125)
    h1, alsd1 = _l1prep(xp, xst, gamma_n, beta_n, W1, AS1)
    ea1, sp1 = _pass_a(alsd1, src2, dst2, z4)
    s1 = sp1[0] + sp1[1]
    acc1 = _pass_b(False, ea1, s1, h1, h1, src2, dst2, z16)

    h2lo, h2hi, alsd2 = _lprep(2, "sum", acc1, b1, Wp2, ASp2)
    ea2, sp2 = _pass_a(alsd2, src2, dst2, z4)
    s2 = sp2[0] + sp2[1]
    acc2 = _pass_b(True, ea2, s2, h2lo, h2hi, src2, dst2, z16)

    q0, q1, q2, q3, alsd3 = _lprep(4, "concat", acc2, b2, Wp3, ASp3)
    ea3, sp3 = _pass_a(alsd3, src2, dst2, z4)
    s3 = sp3[0] + sp3[1]
    acc3a = _pass_b(True, ea3, s3, q0, q1, src2, dst2, z16)
    acc3b = _pass_b(True, ea3, s3, q2, q3, src2, dst2, z16)

    h3f = _mlpprep(acc3a, acc3b, b3)
    gs, gd = _mlp_gather(h3f, srcm, dstm)

    emu = est[0] * (1.0 / E)
    evar = est[1] * (1.0 / E) - emu * emu
    scale = gamma_e / jnp.sqrt(evar + 1e-5)
    A = Wm1[0:64]
    Bm = Wm1[64:128]
    Cp = Wm1[128:138] * scale[:, None]
    bp = bm1 + (beta_e - emu * scale) @ Wm1[128:138]
    return _mlp(gs[:E], gd[:E], e, A, Bm, Cp, bp, Wm2, bm2, Wm3, bm3)


# speedup vs baseline: 37.0383x; 1.1138x over previous
import functools

import jax
import jax.numpy as jnp
from jax import lax
from jax.experimental import pallas as pl
from jax.experimental.pallas import tpu as pltpu
from jax.experimental.pallas import tpu_sc as plsc

N = 50000
E = 800000
H = 3

NP = 50048
CH = NP // 16
E2 = E + N
B = 128
EPT = 26624
E2P = EPT * 32
NBLK = EPT // B
EPT3 = EPT * 2
NBLK3 = NBLK * 2
EPTM = 25088
EMP = EPTM * 32
NBLKM = EPTM // B
SBK = 4
SB = SBK * B
SBKA = 8
SBA = SBKA * B
SBKM = 4
SBM = SBKM * B

@functools.cache
def _mesh_kw():
    return dict(mesh=plsc.VectorSubcoreMesh(core_axis_name="c",
                                            subcore_axis_name="s"),
                compiler_params=pltpu.CompilerParams(needs_layout_passes=False,
                                                     use_tc_tiling_on_sc=False))
_f32 = jnp.float32
_i32 = jnp.int32


def _leaky(x, slope):
    return jnp.where(x >= 0, x, x * slope)


def _iota16():
    return lax.iota(_i32, 16)


def _c16(v, dtype=_i32):
    return jnp.full((16,), v, dtype)


def _pass_a_body(alsd, src, dst, zrow, ea_out, s_out, srcv, dstv, asrc, adst,
                 eav, acc, sem):
    c = lax.axis_index("c")
    s = lax.axis_index("s")
    wid = c * 16 + s
    pltpu.sync_copy(zrow, acc.at[pl.ds(s * CH, CH)])
    for g in range(SBA // 16):
        plsc.store_scatter(eav, [_iota16() + g * 16, _c16(3)],
                           jnp.zeros((16,), _f32))
    plsc.subcore_barrier()

    base = wid * EPT
    base_r = wid * NBLK

    def blk(i, carry):
        off = base + i * SBA
        row0 = base_r + i * SBKA
        pltpu.sync_copy(src.at[pl.ds(row0, SBKA)], srcv)
        pltpu.sync_copy(dst.at[pl.ds(row0, SBKA)], dstv)
        cps = []
        for k in range(SBKA):
            cps.append(pltpu.async_copy(
                alsd.at[srcv.at[k]], asrc.at[pl.ds(k * B, B)], sem))
            cps.append(pltpu.async_copy(
                alsd.at[dstv.at[k]], adst.at[pl.ds(k * B, B)], sem))
        for cp in cps:
            cp.wait()

        def grp(j, carry2):
            for gg in range(8):
                eid = _iota16() + (j * 8 + gg) * 16
                live = (eid + off) < E2
                for h in range(H):
                    a1 = plsc.load_gather(asrc, [eid, _c16(h)])
                    a2 = plsc.load_gather(adst, [eid, _c16(4 + h)])
                    al = _leaky(a1 + a2, 0.2)
                    ea = jnp.where(live, jnp.exp(al), 0.0)
                    plsc.store_scatter(eav, [eid, _c16(h)], ea)
            return carry2

        lax.fori_loop(0, SBA // B, grp, 0)
        pltpu.sync_copy(eav, ea_out.at[pl.ds(off, SBA)])
        for k in range(SBKA):
            pltpu.sync_copy(eav.at[pl.ds(k * B, B)], acc.at[dstv.at[k]],
                            add=True)
        return carry

    lax.fori_loop(0, EPT // SBA, blk, 0)
    plsc.subcore_barrier()
    lo = s * CH
    pltpu.sync_copy(acc.at[pl.ds(lo, CH)], s_out.at[c, pl.ds(lo, CH)])


def _pass_a(alsd, src, dst, zrow4):
    f = pl.kernel(
        _pass_a_body,
        out_type=[jax.ShapeDtypeStruct((E2P, 4), _f32),
                  jax.ShapeDtypeStruct((2, NP, 4), _f32)],
        scratch_types=[pltpu.VMEM((SBKA, B), _i32), pltpu.VMEM((SBKA, B), _i32),
                       pltpu.VMEM((SBA, 8), _f32), pltpu.VMEM((SBA, 8), _f32),
                       pltpu.VMEM((SBA, 4), _f32),
                       pltpu.VMEM_SHARED((NP, 4), _f32),
                       pltpu.SemaphoreType.DMA],
        **_mesh_kw())
    return f(alsd, src, dst, zrow4)


def _pass_b_body(col_split, ea_in, s_in, htab0, htab1, src, dst, zrow,
                 acc_out, srcv, dstv, eav, srow, hrows, outr, acc, sem0, sem1):
    c = lax.axis_index("c")
    s = lax.axis_index("s")
    pltpu.sync_copy(zrow, acc.at[pl.ds(s * CH, CH)])
    plsc.subcore_barrier()

    if col_split:
        base = s * EPT3
        base_r = s * (NBLK * 2)
        nsb = EPT3 // SB
    else:
        base = (c * 16 + s) * EPT
        base_r = (c * 16 + s) * NBLK
        nsb = EPT // SB

    sems = [sem0, sem1]

    def stage(htab, sb, b):
        row0 = base_r + sb * SBK
        pltpu.sync_copy(src.at[pl.ds(row0, SBK)], srcv.at[b])
        pltpu.sync_copy(dst.at[pl.ds(row0, SBK)], dstv.at[b])
        pltpu.sync_copy(ea_in.at[pl.ds((base + sb * SB), SB)], eav.at[b])
        for k in range(SBK):
            pltpu.async_copy(s_in.at[dstv.at[b, k]],
                             srow.at[b, pl.ds(k * B, B)], sems[b])
            pltpu.async_copy(htab.at[srcv.at[b, k]],
                             hrows.at[b, pl.ds(k * B, B)], sems[b])

    def drain(htab, b):
        for k in range(SBK):
            pltpu.make_async_copy(s_in.at[dstv.at[b, k]],
                                  srow.at[b, pl.ds(k * B, B)], sems[b]).wait()
            pltpu.make_async_copy(htab.at[srcv.at[b, k]],
                                  hrows.at[b, pl.ds(k * B, B)], sems[b]).wait()

    def run(htab):
        stage(htab, 0, 0)
        stage(htab, 1, 1)

        def blk(i, carry):
            for b in range(2):
                sb = i * 2 + b
                drain(htab, b)

                def grp(j, carry2):
                    for gg in range(4):
                        eid = _iota16() + (j * 4 + gg) * 16
                        coef = []
                        for h in range(H):
                            eh = plsc.load_gather(eav.at[b], [eid, _c16(h)])
                            sh = plsc.load_gather(srow.at[b], [eid, _c16(h)])
                            coef.append(eh / (sh + 1e-16))
                        for col in range(16):
                            v = coef[0] * plsc.load_gather(
                                hrows.at[b], [eid, _c16(col)])
                            v += coef[1] * plsc.load_gather(
                                hrows.at[b], [eid, _c16(16 + col)])
                            v += coef[2] * plsc.load_gather(
                                hrows.at[b], [eid, _c16(32 + col)])
                            plsc.store_scatter(outr, [eid, _c16(col)], v)
                    return carry2

                lax.fori_loop(0, SB // 64, grp, 0)
                for k in range(SBK):
                    pltpu.sync_copy(outr.at[pl.ds(k * B, B)],
                                    acc.at[dstv.at[b, k]], add=True)

                @pl.when(sb + 2 < nsb)
                def _():
                    stage(htab, sb + 2, b)
            return carry

        lax.fori_loop(0, nsb // 2, blk, 0)

    @pl.when(c == 0)
    def _():
        run(htab0)

    @pl.when(c == 1)
    def _():
        run(htab1 if col_split else htab0)

    plsc.subcore_barrier()
    lo = s * CH
    pltpu.sync_copy(acc.at[pl.ds(lo, CH)], acc_out.at[c, pl.ds(lo, CH)])


def _pass_b(col_split, ea, s_comb, htab0, htab1, src, dst, zrow):
    body = functools.partial(_pass_b_body, col_split)
    f = pl.kernel(
        body,
        out_type=jax.ShapeDtypeStruct((2, NP, 16), _f32),
        scratch_types=[pltpu.VMEM((2, SBK, B), _i32),
                       pltpu.VMEM((2, SBK, B), _i32),
                       pltpu.VMEM((2, SB, 4), _f32),
                       pltpu.VMEM((2, SB, 4), _f32),
                       pltpu.VMEM((2, SB, 48), _f32),
                       pltpu.VMEM((SB, 16), _f32),
                       pltpu.VMEM_SHARED((NP, 16), _f32),
                       pltpu.SemaphoreType.DMA, pltpu.SemaphoreType.DMA],
        **_mesh_kw())
    return f(ea, s_comb, htab0, htab1, src, dst, zrow)


def _mlp_gather_body(h3f, srcm, dstm, gs_out, gd_out, idxs, idxd, rows, sem):
    wid = lax.axis_index("c") * 16 + lax.axis_index("s")
    base = wid * EPTM
    base_r = wid * NBLKM

    def blk(i, carry):
        off = base + i * SBM
        row0 = base_r + i * SBKM
        pltpu.sync_copy(srcm.at[pl.ds(row0, SBKM)], idxs)
        pltpu.sync_copy(dstm.at[pl.ds(row0, SBKM)], idxd)
        cps = [pltpu.async_copy(h3f.at[idxs.at[k]],
                                rows.at[pl.ds(k * B, B)], sem)
               for k in range(SBKM)]
        for cp in cps:
            cp.wait()
        pltpu.sync_copy(rows, gs_out.at[pl.ds(off, SBM)])
        cps = [pltpu.async_copy(h3f.at[idxd.at[k]],
                                rows.at[pl.ds(k * B, B)], sem)
               for k in range(SBKM)]
        for cp in cps:
            cp.wait()
        pltpu.sync_copy(rows, gd_out.at[pl.ds(off, SBM)])
        return carry

    lax.fori_loop(0, EPTM // SBM, blk, 0)


def _mlp_gather(h3f, srcm, dstm):
    f = pl.kernel(
        _mlp_gather_body,
        out_type=[jax.ShapeDtypeStruct((EMP, 64), _f32),
                  jax.ShapeDtypeStruct((EMP, 64), _f32)],
        scratch_types=[pltpu.VMEM((SBKM, B), _i32), pltpu.VMEM((SBKM, B), _i32),
                       pltpu.VMEM((SBM, 64), _f32),
                       pltpu.SemaphoreType.DMA],
        **_mesh_kw())
    return f(h3f, srcm, dstm)


def _stats_body(x_ref, out_ref):
    @pl.when(pl.program_id(0) == 0)
    def _():
        out_ref[...] = jnp.zeros_like(out_ref)
    xv = x_ref[...]
    out_ref[0, :] += jnp.sum(xv, axis=0)
    out_ref[1, :] += jnp.sum(xv * xv, axis=0)


def _stats(x, grid):
    n, d = x.shape
    rb = n // grid
    return pl.pallas_call(
        _stats_body,
        grid=(grid,),
        in_specs=[pl.BlockSpec((rb, d), lambda i: (i, 0))],
        out_specs=pl.BlockSpec((2, d), lambda i: (0, 0)),
        out_shape=jax.ShapeDtypeStruct((2, d), _f32),
    )(x)


def _l1prep_body(x_ref, st_ref, gn_ref, bn_ref, w_ref, as_ref, h_ref, al_ref):
    st = st_ref[...]
    mu = st[0:1] * (1.0 / N)
    var = st[1:2] * (1.0 / N) - mu * mu
    xb = (x_ref[...] - mu) / jnp.sqrt(var + 1e-5) * gn_ref[...] + bn_ref[...]
    h = jnp.dot(xb, w_ref[...], preferred_element_type=_f32)
    h_ref[...] = h
    al_ref[...] = jnp.dot(h, as_ref[...], preferred_element_type=_f32)


def _l1prep(xp, xst, gn, bn, W1, AS1):
    grid = 8
    rb = NP // grid
    full = lambda *shape: pl.BlockSpec(shape, lambda i: tuple(0 for _ in shape))
    return pl.pallas_call(
        _l1prep_body,
        grid=(grid,),
        in_specs=[pl.BlockSpec((rb, 16), lambda i: (i, 0)),
                  full(2, 16), full(16), full(16),
                  full(16, 48), full(48, 8)],
        out_specs=[pl.BlockSpec((rb, 48), lambda i: (i, 0)),
                   pl.BlockSpec((rb, 8), lambda i: (i, 0))],
        out_shape=[jax.ShapeDtypeStruct((NP, 48), _f32),
                   jax.ShapeDtypeStruct((NP, 8), _f32)],
    )(xp, xst, gn, bn, W1, AS1)


def _lprep_body(nsplit, combine, acc_ref, b_ref, w_ref, as_ref, *out_refs):
    a = acc_ref[...]
    if combine == "sum":
        prev = (a[0] + a[1]) * (1.0 / H) + b_ref[...]
    else:
        prev = jnp.concatenate([a[0], a[1]], 1) * (1.0 / H) + b_ref[...]
    h = jnp.dot(prev, w_ref[...], preferred_element_type=_f32)
    out_refs[-1][...] = jnp.dot(h, as_ref[...], preferred_element_type=_f32)
    for q in range(nsplit):
        out_refs[q][...] = h[:, 48 * q:48 * (q + 1)]


def _lprep(nsplit, combine, acc, b, Wp, ASp):
    ci = Wp.shape[0]
    body = functools.partial(_lprep_body, nsplit, combine)
    grid = 8
    rb = NP // grid
    full = lambda *shape: pl.BlockSpec(shape, lambda i: tuple(0 for _ in shape))
    ci_in = acc.shape[2]
    return pl.pallas_call(
        body,
        grid=(grid,),
        in_specs=[pl.BlockSpec((2, rb, ci_in), lambda i: (0, i, 0)),
                  full(ci), full(ci, 48 * nsplit), full(48 * nsplit, 8)],
        out_specs=[pl.BlockSpec((rb, 48), lambda i: (i, 0))] * nsplit
                  + [pl.BlockSpec((rb, 8), lambda i: (i, 0))],
        out_shape=[jax.ShapeDtypeStruct((NP, 48), _f32)] * nsplit
                  + [jax.ShapeDtypeStruct((NP, 8), _f32)],
    )(acc, b, Wp, ASp)


def _mlpprep_body(acca_ref, accb_ref, b_ref, out_ref):
    a = acca_ref[...]
    bb = accb_ref[...]
    out_ref[...] = (jnp.concatenate([a[0], a[1], bb[0], bb[1]], 1)
                    * (1.0 / H) + b_ref[...])


def _mlpprep(acc3a, acc3b, b3):
    grid = 8
    rb = NP // grid
    full = lambda *shape: pl.BlockSpec(shape, lambda i: tuple(0 for _ in shape))
    return pl.pallas_call(
        _mlpprep_body,
        grid=(grid,),
        in_specs=[pl.BlockSpec((2, rb, 16), lambda i: (0, i, 0)),
                  pl.BlockSpec((2, rb, 16), lambda i: (0, i, 0)), full(64)],
        out_specs=pl.BlockSpec((rb, 64), lambda i: (i, 0)),
        out_shape=jax.ShapeDtypeStruct((NP, 64), _f32),
    )(acc3a, acc3b, b3)


def _mlp_body(gs_ref, gd_ref, e_ref, a_ref, b_ref, c_ref, bp_ref, w2_ref,
              b2_ref, w3_ref, b3_ref, out_ref):
    z = (jnp.dot(gs_ref[...], a_ref[...], preferred_element_type=_f32)
         + jnp.dot(gd_ref[...], b_ref[...], preferred_element_type=_f32)
         + jnp.dot(e_ref[...], c_ref[...], preferred_element_type=_f32)
         + bp_ref[...])
    z = _leaky(z, 0.12)
    z = jnp.dot(z, w2_ref[...], preferred_element_type=_f32) + b2_ref[...]
    z = _leaky(z, 0.12)
    z = jnp.dot(z, w3_ref[...], preferred_element_type=_f32) + b3_ref[...]
    out_ref[...] = 1.0 / (1.0 + jnp.exp(-z))


def _mlp(gs, gd, e, A, Bm, Cp, bp, Wm2, bm2, Wm3, bm3):
    grid = 125
    rb = E // grid
    full = lambda *shape: pl.BlockSpec(shape, lambda i: tuple(0 for _ in shape))
    return pl.pallas_call(
        _mlp_body,
        grid=(grid,),
        in_specs=[pl.BlockSpec((rb, 64), lambda i: (i, 0)),
                  pl.BlockSpec((rb, 64), lambda i: (i, 0)),
                  pl.BlockSpec((rb, 10), lambda i: (i, 0)),
                  full(64, 64), full(64, 64), full(10, 64), full(64),
                  full(64, 16), full(16), full(16, 1), full(1)],
        out_specs=pl.BlockSpec((rb, 1), lambda i: (i, 0)),
        out_shape=jax.ShapeDtypeStruct((E, 1), _f32),
    )(gs, gd, e, A, Bm, Cp, bp, Wm2, bm2, Wm3, bm3)


def _perm(co):
    idx = [h * co + q * 16 + j
           for q in range(co // 16) for h in range(H) for j in range(16)]
    return jnp.array(idx, _i32)


def _make_as(a_s, a_d, co):
    AS = jnp.zeros((H * co, 8), _f32)
    for h in range(H):
        AS = AS.at[h * co:(h + 1) * co, h].set(a_s[h])
        AS = AS.at[h * co:(h + 1) * co, 4 + h].set(a_d[h])
    return AS


def kernel(x, edge_index, e, xbatch, gamma_n, beta_n, gamma_e, beta_e,
           W1, as1, ad1, b1, W2, as2, ad2, b2, W3, as3, ad3, b3,
           Wm1, bm1, Wm2, bm2, Wm3, bm3):
    src = edge_index[0]
    dst = edge_index[1]
    loop = jnp.arange(N, dtype=_i32)
    padg = jnp.full((E2P - E2,), N, _i32)
    src2 = jnp.concatenate([src, loop, padg]).reshape(E2P // B, B)
    dst2 = jnp.concatenate([dst, loop, padg]).reshape(E2P // B, B)
    padm = jnp.full((EMP - E,), N, _i32)
    srcm = jnp.concatenate([src, padm]).reshape(EMP // B, B)
    dstm = jnp.concatenate([dst, padm]).reshape(EMP // B, B)

    z4 = jnp.zeros((CH, 4), _f32)
    z16 = jnp.zeros((CH, 16), _f32)

    AS1 = _make_as(as1, ad1, 16)
    p2 = _perm(32)
    Wp2 = jnp.take(W2, p2, axis=1)
    ASp2 = jnp.take(_make_as(as2, ad2, 32), p2, axis=0)
    p3 = _perm(64)
    Wp3 = jnp.take(W3, p3, axis=1)
    ASp3 = jnp.take(_make_as(as3, ad3, 64), p3, axis=0)

    xp = jnp.concatenate([x, jnp.zeros((NP - N, 16), _f32)])
    xst = _stats(x, 10)
    est = _stats(e, 125)
    h1, alsd1 = _l1prep(xp, xst, gamma_n, beta_n, W1, AS1)
    ea1, sp1 = _pass_a(alsd1, src2, dst2, z4)
    s1 = sp1[0] + sp1[1]
    acc1 = _pass_b(False, ea1, s1, h1, h1, src2, dst2, z16)

    h2lo, h2hi, alsd2 = _lprep(2, "sum", acc1, b1, Wp2, ASp2)
    ea2, sp2 = _pass_a(alsd2, src2, dst2, z4)
    s2 = sp2[0] + sp2[1]
    acc2 = _pass_b(True, ea2, s2, h2lo, h2hi, src2, dst2, z16)

    q0, q1, q2, q3, alsd3 = _lprep(4, "concat", acc2, b2, Wp3, ASp3)
    ea3, sp3 = _pass_a(alsd3, src2, dst2, z4)
    s3 = sp3[0] + sp3[1]
    acc3a = _pass_b(True, ea3, s3, q0, q1, src2, dst2, z16)
    acc3b = _pass_b(True, ea3, s3, q2, q3, src2, dst2, z16)

    h3f = _mlpprep(acc3a, acc3b, b3)
    gs, gd = _mlp_gather(h3f, srcm, dstm)

    emu = est[0] * (1.0 / E)
    evar = est[1] * (1.0 / E) - emu * emu
    scale = gamma_e / jnp.sqrt(evar + 1e-5)
    A = Wm1[0:64]
    Bm = Wm1[64:128]
    Cp = Wm1[128:138] * scale[:, None]
    bp = bm1 + (beta_e - emu * scale) @ Wm1[128:138]
    return _mlp(gs[:E], gd[:E], e, A, Bm, Cp, bp, Wm2, bm2, Wm3, bm3)
